# baseline (speedup 1.0000x reference)
import jax
import jax.numpy as jnp
from jax import lax
from jax.experimental import pallas as pl
from jax.experimental.pallas import tpu as pltpu
from jax.experimental.pallas import tpu_sc as plsc

_VOCAB = 1000000
_HIDDEN = 64
_PAD = 128
_BATCH = 16384
_HIST = 200

_NC = 2
_NS = 16
_NW = _NC * _NS
_RPW = _BATCH // _NW


def _body(x_hbm, w_hbm, out_hbm, idx_v, g_v, c_v, sg, sw):
    wid = lax.axis_index("s") * _NC + lax.axis_index("c")
    base = wid * _RPW

    def row(b, carry):
        pltpu.sync_copy(x_hbm.at[pl.ds(b * _HIST, _HIST)], idx_v)
        pltpu.async_copy(w_hbm.at[idx_v], g_v, sg).wait()

        def compact(i, c2):
            src = g_v.at[i]
            dst = c_v.at[i]
            for k in range(_HIDDEN // 16):
                dst[pl.ds(k * 16, 16)] = src[pl.ds(k * 16, 16)]
            return c2

        lax.fori_loop(0, _HIST, compact, 0)
        pltpu.sync_copy(c_v, out_hbm.at[b])
        return carry

    lax.fori_loop(base, base + _RPW, row, 0)


def kernel(x, weight):
    xf = x.reshape(-1).astype(jnp.int32)
    wp = jnp.pad(weight, ((0, 0), (0, _PAD - _HIDDEN)))
    mesh = plsc.VectorSubcoreMesh(
        core_axis_name="c", subcore_axis_name="s",
        num_cores=_NC, num_subcores=_NS)
    out = pl.kernel(
        _body,
        out_type=jax.ShapeDtypeStruct((_BATCH, _HIST, _HIDDEN), jnp.float32),
        mesh=mesh,
        compiler_params=pltpu.CompilerParams(use_tc_tiling_on_sc=True),
        scratch_types=[
            pltpu.VMEM((_HIST,), jnp.int32),
            pltpu.VMEM((_HIST, _PAD), jnp.float32),
            pltpu.VMEM((_HIST, _HIDDEN), jnp.float32),
            pltpu.SemaphoreType.DMA,
            pltpu.SemaphoreType.DMA,
        ],
    )(xf, wp)
    return out
